# baseline (speedup 1.0000x reference)
import functools

import jax
import jax.numpy as jnp
from jax import lax
from jax.experimental import pallas as pl
from jax.experimental.pallas import tpu as pltpu
from jax.experimental.pallas import tpu_sc as plsc

L = 1024
VOCA_DIM = 64
ADD_DIM = 32
EMBED_DIM = VOCA_DIM + ADD_DIM
VOCAB = 100000
PACK_W = 128


_PT = 4096
_PG = pl.cdiv(VOCAB, _PT)
_PTAIL = VOCAB - (_PG - 1) * _PT
_PNBUF = 4


_RCH_E = 8
_RCH_S = 4


def _pack_body(embT_hbm, synT_hbm, out_hbm, embv, synv, bufs, rsems, sems):
    i = pl.program_id(0)
    slot = lax.rem(i, _PNBUF)

    @pl.when(i == 0)
    def _read_tables():
        for r in range(_RCH_E):
            pltpu.make_async_copy(
                embT_hbm.at[pl.ds(r * 8, 8)], embv.at[pl.ds(r * 8, 8)],
                rsems.at[r],
            ).start()
        for r in range(_RCH_S):
            pltpu.make_async_copy(
                synT_hbm.at[pl.ds(r * 8, 8)], synv.at[pl.ds(r * 8, 8)],
                rsems.at[_RCH_E + r],
            ).start()
        for r in range(_RCH_E):
            pltpu.make_async_copy(
                embT_hbm.at[pl.ds(r * 8, 8)], embv.at[pl.ds(r * 8, 8)],
                rsems.at[r],
            ).wait()
        for r in range(_RCH_S):
            pltpu.make_async_copy(
                synT_hbm.at[pl.ds(r * 8, 8)], synv.at[pl.ds(r * 8, 8)],
                rsems.at[_RCH_E + r],
            ).wait()

    @pl.when(i >= _PNBUF)
    def _wait_slot():
        pltpu.make_async_copy(
            bufs.at[slot], out_hbm.at[pl.ds((i - _PNBUF) * _PT, _PT)],
            sems.at[slot],
        ).wait()

    col = pl.multiple_of(i * _PT, 128)
    z = jnp.zeros((_PT, PACK_W - EMBED_DIM), jnp.float32)

    @pl.when(i < _PG - 1)
    def _full_tile():
        emb = embv[:, pl.ds(col, _PT)].T
        syn = synv[:, pl.ds(col, _PT)].T
        bufs[slot] = jnp.concatenate([emb, syn, z], axis=1)

    @pl.when(i == _PG - 1)
    def _tail_tile():
        emb = embv[:, pl.ds((_PG - 1) * _PT, _PTAIL)].T
        syn = synv[:, pl.ds((_PG - 1) * _PT, _PTAIL)].T
        zt = jnp.zeros((_PTAIL, PACK_W - EMBED_DIM), jnp.float32)
        bufs[slot, pl.ds(0, _PTAIL)] = jnp.concatenate([emb, syn, zt], axis=1)

    @pl.when(i < _PG - 1)
    def _start_full():
        pltpu.make_async_copy(
            bufs.at[slot], out_hbm.at[pl.ds(i * _PT, _PT)], sems.at[slot],
        ).start()

    @pl.when(i == _PG - 1)
    def _tail_and_drain():
        pltpu.make_async_copy(
            bufs.at[slot, pl.ds(0, _PTAIL)],
            out_hbm.at[pl.ds((_PG - 1) * _PT, _PTAIL)],
            sems.at[slot],
        ).start()
        for k in range(1, _PNBUF):
            s = (_PG - 1 - k) % _PNBUF
            pltpu.make_async_copy(
                bufs.at[s], out_hbm.at[pl.ds(0, _PT)], sems.at[s],
            ).wait()
        pltpu.make_async_copy(
            bufs.at[slot, pl.ds(0, _PTAIL)],
            out_hbm.at[pl.ds(0, _PTAIL)],
            sems.at[slot],
        ).wait()


def _tc_pack(embT, synT):
    return pl.pallas_call(
        _pack_body,
        grid=(_PG,),
        in_specs=[
            pl.BlockSpec(memory_space=pl.ANY),
            pl.BlockSpec(memory_space=pl.ANY),
        ],
        out_specs=pl.BlockSpec(memory_space=pl.ANY),
        out_shape=jax.ShapeDtypeStruct((VOCAB, PACK_W), jnp.float32),
        scratch_shapes=[
            pltpu.VMEM((VOCA_DIM, VOCAB), jnp.float32),
            pltpu.VMEM((ADD_DIM, VOCAB), jnp.float32),
            pltpu.VMEM((_PNBUF, _PT, PACK_W), jnp.float32),
            pltpu.SemaphoreType.DMA((_RCH_E + _RCH_S,)),
            pltpu.SemaphoreType.DMA((_PNBUF,)),
        ],
    )(embT, synT)


_info = plsc.get_sparse_core_info()
_NC, _NS = _info.num_cores, _info.num_subcores
_NW = _NC * _NS
_B_PER_W = L // _NW


def _sc_gather(ids, packed):
    mesh = plsc.VectorSubcoreMesh(core_axis_name="c", subcore_axis_name="s")

    @functools.partial(
        pl.kernel,
        mesh=mesh,
        out_type=jax.ShapeDtypeStruct((L, PACK_W), jnp.float32),
        scratch_types=[
            pltpu.VMEM((_B_PER_W,), jnp.int32),
            pltpu.VMEM((_B_PER_W, PACK_W), jnp.float32),
            pltpu.SemaphoreType.DMA,
        ],
    )
    def gather_kernel(ids_hbm, tab_hbm, out_hbm, idx_v, rows_v, sem):
        wid = lax.axis_index("s") * _NC + lax.axis_index("c")
        base = wid * _B_PER_W
        pltpu.sync_copy(ids_hbm.at[pl.ds(base, _B_PER_W)], idx_v)
        pltpu.async_copy(tab_hbm.at[idx_v], rows_v, sem).wait()
        pltpu.sync_copy(rows_v, out_hbm.at[pl.ds(base, _B_PER_W)])

    return gather_kernel(ids, packed)


_VT = 2048
_G = pl.cdiv(VOCAB, _VT)
_VTAIL = VOCAB - (_G - 1) * _VT
_NBUF = 4


def _mm_body(rows_ref, synw_ref, padT_ref, revT_ref, out_hbm, bufs, sems):
    i = pl.program_id(0)
    slot = lax.rem(i, _NBUF)

    @pl.when(i >= _NBUF)
    def _wait_slot():
        pltpu.make_async_copy(
            bufs.at[slot], out_hbm.at[pl.ds((i - _NBUF) * _VT, _VT)],
            sems.at[slot],
        ).wait()

    rowsT = rows_ref[...].T
    embT = rowsT[:VOCA_DIM, :]
    synT = rowsT[VOCA_DIM:EMBED_DIM, :]
    projT = lax.dot_general(
        synw_ref[...], synT,
        dimension_numbers=(((0,), (0,)), ((), ())),
        preferred_element_type=jnp.float32,
    )
    xT = jnp.concatenate([embT + projT, padT_ref[...]], axis=0)
    bufs[slot] = lax.dot_general(
        revT_ref[...], xT,
        dimension_numbers=(((0,), (0,)), ((), ())),
        preferred_element_type=jnp.float32,
    )

    @pl.when(i < _G - 1)
    def _start_full():
        pltpu.make_async_copy(
            bufs.at[slot], out_hbm.at[pl.ds(i * _VT, _VT)], sems.at[slot],
        ).start()

    @pl.when(i == _G - 1)
    def _tail_and_drain():
        pltpu.make_async_copy(
            bufs.at[slot, pl.ds(0, _VTAIL)],
            out_hbm.at[pl.ds((_G - 1) * _VT, _VTAIL)],
            sems.at[slot],
        ).start()
        for k in range(1, _NBUF):
            s = (_G - 1 - k) % _NBUF
            pltpu.make_async_copy(
                bufs.at[s], out_hbm.at[pl.ds(0, _VT)], sems.at[s],
            ).wait()
        pltpu.make_async_copy(
            bufs.at[slot, pl.ds(0, _VTAIL)],
            out_hbm.at[pl.ds(0, _VTAIL)],
            sems.at[slot],
        ).wait()


def _tc_matmul(rows, syn_weight, padT, revT):
    return pl.pallas_call(
        _mm_body,
        grid=(_G,),
        in_specs=[
            pl.BlockSpec((L, PACK_W), lambda i: (0, 0)),
            pl.BlockSpec((ADD_DIM, VOCA_DIM), lambda i: (0, 0)),
            pl.BlockSpec((ADD_DIM, L), lambda i: (0, 0)),
            pl.BlockSpec((EMBED_DIM, _VT), lambda i: (0, i)),
        ],
        out_specs=pl.BlockSpec(memory_space=pl.ANY),
        out_shape=jax.ShapeDtypeStruct((VOCAB, L), jnp.float32),
        scratch_shapes=[
            pltpu.VMEM((_NBUF, _VT, L), jnp.float32),
            pltpu.SemaphoreType.DMA((_NBUF,)),
        ],
        compiler_params=pltpu.CompilerParams(
            fuse_transposed_lhs_in_matmul=True,
        ),
    )(rows, syn_weight, padT, revT)


def kernel(ids, emb_weight, to_syn_weight, syn_weight, rev_weight, padding):
    packed = _tc_pack(emb_weight.T, to_syn_weight.T)
    rows = _sc_gather(ids, packed)
    outT = _tc_matmul(rows, syn_weight, padding[:L, :].T, rev_weight.T)
    return outT.T


# speedup vs baseline: 1.0310x; 1.0310x over previous
import functools

import jax
import jax.numpy as jnp
from jax import lax
from jax.experimental import pallas as pl
from jax.experimental.pallas import tpu as pltpu
from jax.experimental.pallas import tpu_sc as plsc

L = 1024
VOCA_DIM = 64
ADD_DIM = 32
EMBED_DIM = VOCA_DIM + ADD_DIM
VOCAB = 100000
PACK_W = 128


_PT = 4096
_PG = pl.cdiv(VOCAB, _PT)
_PTAIL = VOCAB - (_PG - 1) * _PT
_SLIV = _PTAIL % 128
_PTA = _PTAIL - _SLIV
_PNBUF = 4


_RNBUF = 4


def _start_reads(embT_hbm, synT_hbm, embi, syni, tail_e, tail_s,
                 rsems_e, rsems_s, tsem_e, tsem_s, j):
    rslot = lax.rem(j, _RNBUF)

    col = pl.multiple_of(j * _PT, 128)

    @pl.when(j < _PG - 1)
    def _full():
        pltpu.make_async_copy(
            embT_hbm.at[:, pl.ds(col, _PT)], embi.at[rslot],
            rsems_e.at[rslot],
        ).start()
        pltpu.make_async_copy(
            synT_hbm.at[:, pl.ds(col, _PT)], syni.at[rslot],
            rsems_s.at[rslot],
        ).start()

    @pl.when(j == _PG - 1)
    def _tail():
        pltpu.make_async_copy(
            embT_hbm.at[:, pl.ds((_PG - 1) * _PT, _PTA)], tail_e, tsem_e,
        ).start()
        pltpu.make_async_copy(
            synT_hbm.at[:, pl.ds((_PG - 1) * _PT, _PTA)], tail_s, tsem_s,
        ).start()


def _pack_body(embT_hbm, synT_hbm, sliv_e_ref, sliv_s_ref, out_hbm,
               embi, syni, tail_e, tail_s, bufs,
               rsems_e, rsems_s, tsem_e, tsem_s, sems):
    i = pl.program_id(0)
    slot = lax.rem(i, _PNBUF)
    rslot = lax.rem(i, _RNBUF)

    @pl.when(i == 0)
    def _prime():
        for j in range(_RNBUF):
            _start_reads(embT_hbm, synT_hbm, embi, syni, tail_e, tail_s,
                         rsems_e, rsems_s, tsem_e, tsem_s, jnp.int32(j))

    @pl.when(i >= _PNBUF)
    def _wait_slot():
        pltpu.make_async_copy(
            bufs.at[slot], out_hbm.at[pl.ds((i - _PNBUF) * _PT, _PT)],
            sems.at[slot],
        ).wait()

    z = jnp.zeros((_PT, PACK_W - EMBED_DIM), jnp.float32)

    @pl.when(i < _PG - 1)
    def _full_tile():
        pltpu.make_async_copy(
            embT_hbm.at[:, pl.ds(0, _PT)], embi.at[rslot], rsems_e.at[rslot],
        ).wait()
        pltpu.make_async_copy(
            synT_hbm.at[:, pl.ds(0, _PT)], syni.at[rslot], rsems_s.at[rslot],
        ).wait()
        bufs[slot] = jnp.concatenate(
            [embi[rslot].T, syni[rslot].T, z], axis=1)

    @pl.when(i == _PG - 1)
    def _tail_tile():
        pltpu.make_async_copy(
            embT_hbm.at[:, pl.ds(0, _PTA)], tail_e, tsem_e,
        ).wait()
        pltpu.make_async_copy(
            synT_hbm.at[:, pl.ds(0, _PTA)], tail_s, tsem_s,
        ).wait()
        zt = jnp.zeros((_PTA, PACK_W - EMBED_DIM), jnp.float32)
        bufs[slot, pl.ds(0, _PTA)] = jnp.concatenate(
            [tail_e[...].T, tail_s[...].T, zt], axis=1)
        zs = jnp.zeros((_SLIV, PACK_W - EMBED_DIM), jnp.float32)
        bufs[slot, pl.ds(_PTA, _SLIV)] = jnp.concatenate(
            [sliv_e_ref[...], sliv_s_ref[...], zs], axis=1)

    @pl.when(i + _RNBUF < _PG)
    def _next_read():
        _start_reads(embT_hbm, synT_hbm, embi, syni, tail_e, tail_s,
                     rsems_e, rsems_s, tsem_e, tsem_s, i + _RNBUF)

    @pl.when(i < _PG - 1)
    def _start_full():
        pltpu.make_async_copy(
            bufs.at[slot], out_hbm.at[pl.ds(i * _PT, _PT)], sems.at[slot],
        ).start()

    @pl.when(i == _PG - 1)
    def _tail_and_drain():
        pltpu.make_async_copy(
            bufs.at[slot, pl.ds(0, _PTAIL)],
            out_hbm.at[pl.ds((_PG - 1) * _PT, _PTAIL)],
            sems.at[slot],
        ).start()
        for k in range(1, _PNBUF):
            s = (_PG - 1 - k) % _PNBUF
            pltpu.make_async_copy(
                bufs.at[s], out_hbm.at[pl.ds(0, _PT)], sems.at[s],
            ).wait()
        pltpu.make_async_copy(
            bufs.at[slot, pl.ds(0, _PTAIL)],
            out_hbm.at[pl.ds(0, _PTAIL)],
            sems.at[slot],
        ).wait()


def _tc_pack(embT, synT, sliv_e, sliv_s):
    return pl.pallas_call(
        _pack_body,
        grid=(_PG,),
        in_specs=[
            pl.BlockSpec(memory_space=pl.ANY),
            pl.BlockSpec(memory_space=pl.ANY),
            pl.BlockSpec((_SLIV, VOCA_DIM), lambda i: (0, 0)),
            pl.BlockSpec((_SLIV, ADD_DIM), lambda i: (0, 0)),
        ],
        out_specs=pl.BlockSpec(memory_space=pl.ANY),
        out_shape=jax.ShapeDtypeStruct((VOCAB, PACK_W), jnp.float32),
        scratch_shapes=[
            pltpu.VMEM((_RNBUF, VOCA_DIM, _PT), jnp.float32),
            pltpu.VMEM((_RNBUF, ADD_DIM, _PT), jnp.float32),
            pltpu.VMEM((VOCA_DIM, _PTA), jnp.float32),
            pltpu.VMEM((ADD_DIM, _PTA), jnp.float32),
            pltpu.VMEM((_PNBUF, _PT, PACK_W), jnp.float32),
            pltpu.SemaphoreType.DMA((_RNBUF,)),
            pltpu.SemaphoreType.DMA((_RNBUF,)),
            pltpu.SemaphoreType.DMA,
            pltpu.SemaphoreType.DMA,
            pltpu.SemaphoreType.DMA((_PNBUF,)),
        ],
    )(embT, synT, sliv_e, sliv_s)


_info = plsc.get_sparse_core_info()
_NC, _NS = _info.num_cores, _info.num_subcores
_NW = _NC * _NS
_B_PER_W = L // _NW


def _sc_gather(ids, packed):
    mesh = plsc.VectorSubcoreMesh(core_axis_name="c", subcore_axis_name="s")

    @functools.partial(
        pl.kernel,
        mesh=mesh,
        out_type=jax.ShapeDtypeStruct((L, PACK_W), jnp.float32),
        scratch_types=[
            pltpu.VMEM((_B_PER_W,), jnp.int32),
            pltpu.VMEM((_B_PER_W, PACK_W), jnp.float32),
            pltpu.SemaphoreType.DMA,
        ],
    )
    def gather_kernel(ids_hbm, tab_hbm, out_hbm, idx_v, rows_v, sem):
        wid = lax.axis_index("s") * _NC + lax.axis_index("c")
        base = wid * _B_PER_W
        pltpu.sync_copy(ids_hbm.at[pl.ds(base, _B_PER_W)], idx_v)
        pltpu.async_copy(tab_hbm.at[idx_v], rows_v, sem).wait()
        pltpu.sync_copy(rows_v, out_hbm.at[pl.ds(base, _B_PER_W)])

    return gather_kernel(ids, packed)


_VT = 2048
_G = pl.cdiv(VOCAB, _VT)
_VTAIL = VOCAB - (_G - 1) * _VT
_NBUF = 4


def _mm_body(rows_ref, synw_ref, padT_ref, revT_ref, out_hbm, bufs, sems):
    i = pl.program_id(0)
    slot = lax.rem(i, _NBUF)

    @pl.when(i >= _NBUF)
    def _wait_slot():
        pltpu.make_async_copy(
            bufs.at[slot], out_hbm.at[pl.ds((i - _NBUF) * _VT, _VT)],
            sems.at[slot],
        ).wait()

    rowsT = rows_ref[...].T
    embT = rowsT[:VOCA_DIM, :]
    synT = rowsT[VOCA_DIM:EMBED_DIM, :]
    projT = lax.dot_general(
        synw_ref[...], synT,
        dimension_numbers=(((0,), (0,)), ((), ())),
        preferred_element_type=jnp.float32,
    )
    xT = jnp.concatenate([embT + projT, padT_ref[...]], axis=0)
    bufs[slot] = lax.dot_general(
        revT_ref[...], xT,
        dimension_numbers=(((0,), (0,)), ((), ())),
        preferred_element_type=jnp.float32,
    )

    @pl.when(i < _G - 1)
    def _start_full():
        pltpu.make_async_copy(
            bufs.at[slot], out_hbm.at[pl.ds(i * _VT, _VT)], sems.at[slot],
        ).start()

    @pl.when(i == _G - 1)
    def _tail_and_drain():
        pltpu.make_async_copy(
            bufs.at[slot, pl.ds(0, _VTAIL)],
            out_hbm.at[pl.ds((_G - 1) * _VT, _VTAIL)],
            sems.at[slot],
        ).start()
        for k in range(1, _NBUF):
            s = (_G - 1 - k) % _NBUF
            pltpu.make_async_copy(
                bufs.at[s], out_hbm.at[pl.ds(0, _VT)], sems.at[s],
            ).wait()
        pltpu.make_async_copy(
            bufs.at[slot, pl.ds(0, _VTAIL)],
            out_hbm.at[pl.ds(0, _VTAIL)],
            sems.at[slot],
        ).wait()


def _tc_matmul(rows, syn_weight, padT, revT):
    return pl.pallas_call(
        _mm_body,
        grid=(_G,),
        in_specs=[
            pl.BlockSpec((L, PACK_W), lambda i: (0, 0)),
            pl.BlockSpec((ADD_DIM, VOCA_DIM), lambda i: (0, 0)),
            pl.BlockSpec((ADD_DIM, L), lambda i: (0, 0)),
            pl.BlockSpec((EMBED_DIM, _VT), lambda i: (0, i)),
        ],
        out_specs=pl.BlockSpec(memory_space=pl.ANY),
        out_shape=jax.ShapeDtypeStruct((VOCAB, L), jnp.float32),
        scratch_shapes=[
            pltpu.VMEM((_NBUF, _VT, L), jnp.float32),
            pltpu.SemaphoreType.DMA((_NBUF,)),
        ],
        compiler_params=pltpu.CompilerParams(
            fuse_transposed_lhs_in_matmul=True,
        ),
    )(rows, syn_weight, padT, revT)


def kernel(ids, emb_weight, to_syn_weight, syn_weight, rev_weight, padding):
    packed = _tc_pack(emb_weight.T, to_syn_weight.T,
                      emb_weight[VOCAB - _SLIV:, :],
                      to_syn_weight[VOCAB - _SLIV:, :])
    rows = _sc_gather(ids, packed)
    outT = _tc_matmul(rows, syn_weight, padding[:L, :].T, rev_weight.T)
    return outT.T
